# Initial kernel scaffold; baseline (speedup 1.0000x reference)
import jax
import jax.numpy as jnp
from jax.experimental import pallas as pl


def kernel(gatheredNodes, elementNodes, elements, elemConnScatter, maxNelem, nodeDim, dimSize, W1, b1, W2, b2, W3, b3, gamma, beta):
    raise NotImplementedError("write your pallas kernel here")


# speedup vs baseline: 2.9788x; 2.9788x over previous
import functools

import jax
import jax.numpy as jnp
from jax import lax
from jax.experimental import pallas as pl
from jax.experimental.pallas import tpu as pltpu
from jax.experimental.pallas import tpu_sc as plsc

NIN = 128
NEN = 16
NEL = 16
HID = 128
E = 100000
MAXN = 4
R = E * MAXN
DIMSIZE = 50000


BR = 4000
N_BLOCKS = R // BR

_INV_SQRT2 = 0.7071067811865476


def _gelu(x):
    return 0.5 * x * (1.0 + lax.erf(x * _INV_SQRT2))


def _mlp_block(gn_ref, ee_ref, w1a_ref, w1bc_ref, b1_ref, w2_ref, b2_ref,
               w3_ref, b3_ref, gm_ref, bt_ref, out_ref):
    x = (jnp.dot(gn_ref[...], w1a_ref[...], preferred_element_type=jnp.float32)
         + jnp.dot(ee_ref[...], w1bc_ref[...], preferred_element_type=jnp.float32)
         + b1_ref[...])
    h = _gelu(x)
    h = _gelu(jnp.dot(h, w2_ref[...], preferred_element_type=jnp.float32)
              + b2_ref[...])
    y = jnp.dot(h, w3_ref[...], preferred_element_type=jnp.float32) + b3_ref[...]
    mu = jnp.mean(y, axis=-1, keepdims=True)
    d = y - mu
    var = jnp.mean(d * d, axis=-1, keepdims=True)
    out_ref[...] = d * lax.rsqrt(var + 1e-5) * gm_ref[...] + bt_ref[...]


def _mlp(gn2, ee, w1a, w1bc, b1, w2, b2, w3, b3, gm, bt):
    row_spec = lambda w: pl.BlockSpec((BR, w), lambda i: (i, 0))
    full_spec = lambda a: pl.BlockSpec(a.shape, lambda i: (0,) * a.ndim)
    return pl.pallas_call(
        _mlp_block,
        grid=(N_BLOCKS,),
        in_specs=[
            row_spec(NIN),
            row_spec(NEN + NEL),
            full_spec(w1a),
            full_spec(w1bc),
            full_spec(b1),
            full_spec(w2),
            full_spec(b2),
            full_spec(w3),
            full_spec(b3),
            full_spec(gm),
            full_spec(bt),
        ],
        out_specs=row_spec(HID),
        out_shape=jax.ShapeDtypeStruct((R, HID), jnp.float32),
        compiler_params=pltpu.CompilerParams(
            dimension_semantics=("arbitrary",),
        ),
    )(gn2, ee, w1a, w1bc, b1, w2, b2, w3, b3, gm, bt)


NCHUNK = 8
CHW = HID // NCHUNK
NTILES = 16
ROWS_PT = R // NTILES
SB = 1000
NB = ROWS_PT // SB
DPT = DIMSIZE // NTILES


def _sc_body(interim, idx3, zrs, out, idx_v, rows_v, acc):
    c = lax.axis_index("c")
    s = lax.axis_index("s")
    for ch in range(NCHUNK):
        f0 = ch * CHW

        @pl.when(c == (ch % 2))
        def _chunk():
            pltpu.sync_copy(zrs, acc.at[pl.ds(s * DPT, DPT)])
            plsc.subcore_barrier()

            def body(b, carry):
                g = s * NB + b
                row0 = g * SB
                pltpu.sync_copy(idx3.at[g], idx_v)
                pltpu.sync_copy(
                    interim.at[pl.ds(row0, SB), pl.ds(f0, CHW)], rows_v)
                for j in range(8):
                    pltpu.sync_copy(
                        rows_v.at[pl.ds(j * 125, 125)],
                        acc.at[idx_v.at[j]],
                        add=True)
                return carry

            lax.fori_loop(0, NB, body, 0)
            plsc.subcore_barrier()
            pltpu.sync_copy(
                acc.at[pl.ds(s * DPT, DPT)],
                out.at[0, pl.ds(s * DPT, DPT), pl.ds(f0, CHW)])
            plsc.subcore_barrier()


def _scatter(interim, idx3, zrs):
    mesh = plsc.VectorSubcoreMesh(core_axis_name="c", subcore_axis_name="s")
    return pl.kernel(
        _sc_body,
        mesh=mesh,
        compiler_params=pltpu.CompilerParams(use_tc_tiling_on_sc=False),
        out_type=jax.ShapeDtypeStruct((1, DIMSIZE, HID), jnp.float32),
        scratch_types=[
            pltpu.VMEM((8, 125), jnp.int32),
            pltpu.VMEM((SB, CHW), jnp.float32),
            pltpu.VMEM_SHARED((DIMSIZE, CHW), jnp.float32),
        ],
    )(interim, idx3, zrs)


def kernel(gatheredNodes, elementNodes, elements, elemConnScatter, maxNelem,
           nodeDim, dimSize, W1, b1, W2, b2, W3, b3, gamma, beta):
    gn2 = gatheredNodes.reshape(R, NIN)
    el_exp = jnp.broadcast_to(elements[:, :, None, :], (1, E, MAXN, NEL))
    ee = jnp.concatenate(
        [elementNodes.reshape(R, NEN), el_exp.reshape(R, NEL)], axis=-1)
    interim = _mlp(gn2, ee, W1[:NIN], W1[NIN:], b1.reshape(1, HID),
                   W2, b2.reshape(1, HID), W3, b3.reshape(1, HID),
                   gamma.reshape(1, HID), beta.reshape(1, HID))
    idx3 = elemConnScatter.reshape(R // SB, 8, 125)
    zrs = jnp.zeros((DPT, CHW), jnp.float32)
    return _scatter(interim, idx3, zrs)


# speedup vs baseline: 3.5949x; 1.2068x over previous
import functools

import jax
import jax.numpy as jnp
from jax import lax
from jax.experimental import pallas as pl
from jax.experimental.pallas import tpu as pltpu
from jax.experimental.pallas import tpu_sc as plsc

NIN = 128
NEN = 16
NEL = 16
HID = 128
E = 100000
MAXN = 4
R = E * MAXN
DIMSIZE = 50000


BR = 4000
N_BLOCKS = R // BR

_INV_SQRT2 = 0.7071067811865476


def _gelu(x):
    return 0.5 * x * (1.0 + lax.erf(x * _INV_SQRT2))


def _mlp_block(gn_ref, ee_ref, w1a_ref, w1bc_ref, b1_ref, w2_ref, b2_ref,
               w3_ref, b3_ref, gm_ref, bt_ref, out_ref):
    x = (jnp.dot(gn_ref[...], w1a_ref[...], preferred_element_type=jnp.float32)
         + jnp.dot(ee_ref[...], w1bc_ref[...], preferred_element_type=jnp.float32)
         + b1_ref[...])
    h = _gelu(x)
    h = _gelu(jnp.dot(h, w2_ref[...], preferred_element_type=jnp.float32)
              + b2_ref[...])
    y = jnp.dot(h, w3_ref[...], preferred_element_type=jnp.float32) + b3_ref[...]
    mu = jnp.mean(y, axis=-1, keepdims=True)
    d = y - mu
    var = jnp.mean(d * d, axis=-1, keepdims=True)
    out_ref[...] = d * lax.rsqrt(var + 1e-5) * gm_ref[...] + bt_ref[...]


def _mlp(gn2, ee, w1a, w1bc, b1, w2, b2, w3, b3, gm, bt):
    row_spec = lambda w: pl.BlockSpec((BR, w), lambda i: (i, 0))
    full_spec = lambda a: pl.BlockSpec(a.shape, lambda i: (0,) * a.ndim)
    return pl.pallas_call(
        _mlp_block,
        grid=(N_BLOCKS,),
        in_specs=[
            row_spec(NIN),
            row_spec(NEN + NEL),
            full_spec(w1a),
            full_spec(w1bc),
            full_spec(b1),
            full_spec(w2),
            full_spec(b2),
            full_spec(w3),
            full_spec(b3),
            full_spec(gm),
            full_spec(bt),
        ],
        out_specs=row_spec(HID),
        out_shape=jax.ShapeDtypeStruct((R, HID), jnp.float32),
        compiler_params=pltpu.CompilerParams(
            dimension_semantics=("arbitrary",),
        ),
    )(gn2, ee, w1a, w1bc, b1, w2, b2, w3, b3, gm, bt)


NCHUNK = 8
CHW = HID // NCHUNK
NTILES = 16
ROWS_PT = R // NTILES
SB = 1000
NB = ROWS_PT // SB
DPT = DIMSIZE // NTILES


def _sc_body(interim, idx3, zrs, out, idx_all, rows_v, acc, sem_in, sem_sc):
    c = lax.axis_index("c")
    s = lax.axis_index("s")
    pltpu.sync_copy(idx3.at[pl.ds(s * NB, NB)], idx_all)

    for ch in range(NCHUNK):
        f0 = ch * CHW

        @pl.when(c == (ch % 2))
        def _chunk():
            pltpu.sync_copy(zrs, acc.at[pl.ds(s * DPT, DPT)])
            plsc.subcore_barrier()

            def _start(b, buf):
                row0 = (s * NB + b) * SB
                pltpu.async_copy(
                    interim.at[pl.ds(row0, SB), pl.ds(f0, CHW)],
                    rows_v.at[buf], sem_in.at[buf])

            _start(0, 0)

            def body(b, carry):
                buf = lax.rem(b, 2)

                @pl.when(b + 1 < NB)
                def _pre():
                    _start(b + 1, 1 - buf)

                pltpu.make_async_copy(
                    interim.at[pl.ds(0, SB), pl.ds(f0, CHW)],
                    rows_v.at[buf], sem_in.at[buf]).wait()
                descs = [
                    pltpu.async_copy(
                        rows_v.at[buf, pl.ds(j * 125, 125)],
                        acc.at[idx_all.at[b, j]],
                        sem_sc, add=True)
                    for j in range(8)
                ]
                for d in descs:
                    d.wait()
                return carry

            lax.fori_loop(0, NB, body, 0)
            plsc.subcore_barrier()
            pltpu.sync_copy(
                acc.at[pl.ds(s * DPT, DPT)],
                out.at[0, pl.ds(s * DPT, DPT), pl.ds(f0, CHW)])
            plsc.subcore_barrier()


def _scatter(interim, idx3, zrs):
    mesh = plsc.VectorSubcoreMesh(core_axis_name="c", subcore_axis_name="s")
    return pl.kernel(
        _sc_body,
        mesh=mesh,
        compiler_params=pltpu.CompilerParams(use_tc_tiling_on_sc=False),
        out_type=jax.ShapeDtypeStruct((1, DIMSIZE, HID), jnp.float32),
        scratch_types=[
            pltpu.VMEM((NB, 8, 125), jnp.int32),
            pltpu.VMEM((2, SB, CHW), jnp.float32),
            pltpu.VMEM_SHARED((DIMSIZE, CHW), jnp.float32),
            pltpu.SemaphoreType.DMA((2,)),
            pltpu.SemaphoreType.DMA,
        ],
    )(interim, idx3, zrs)


def kernel(gatheredNodes, elementNodes, elements, elemConnScatter, maxNelem,
           nodeDim, dimSize, W1, b1, W2, b2, W3, b3, gamma, beta):
    gn2 = gatheredNodes.reshape(R, NIN)
    el_exp = jnp.broadcast_to(elements[:, :, None, :], (1, E, MAXN, NEL))
    ee = jnp.concatenate(
        [elementNodes.reshape(R, NEN), el_exp.reshape(R, NEL)], axis=-1)
    interim = _mlp(gn2, ee, W1[:NIN], W1[NIN:], b1.reshape(1, HID),
                   W2, b2.reshape(1, HID), W3, b3.reshape(1, HID),
                   gamma.reshape(1, HID), beta.reshape(1, HID))
    idx3 = elemConnScatter.reshape(R // SB, 8, 125)
    zrs = jnp.zeros((DPT, CHW), jnp.float32)
    return _scatter(interim, idx3, zrs)


# speedup vs baseline: 3.6057x; 1.0030x over previous
import functools

import jax
import jax.numpy as jnp
from jax import lax
from jax.experimental import pallas as pl
from jax.experimental.pallas import tpu as pltpu
from jax.experimental.pallas import tpu_sc as plsc

NIN = 128
NEN = 16
NEL = 16
HID = 128
E = 100000
MAXN = 4
R = E * MAXN
DIMSIZE = 50000


BR = 4000
N_BLOCKS = R // BR

_INV_SQRT2 = 0.7071067811865476


def _gelu(x):
    return 0.5 * x * (1.0 + lax.erf(x * _INV_SQRT2))


def _mlp_block(gn_ref, ee_ref, w1a_ref, w1bc_ref, b1_ref, w2_ref, b2_ref,
               w3_ref, b3_ref, gm_ref, bt_ref, out_ref):
    x = (jnp.dot(gn_ref[...], w1a_ref[...], preferred_element_type=jnp.float32)
         + jnp.dot(ee_ref[...], w1bc_ref[...], preferred_element_type=jnp.float32)
         + b1_ref[...])
    h = _gelu(x)
    h = _gelu(jnp.dot(h, w2_ref[...], preferred_element_type=jnp.float32)
              + b2_ref[...])
    y = jnp.dot(h, w3_ref[...], preferred_element_type=jnp.float32) + b3_ref[...]
    mu = jnp.mean(y, axis=-1, keepdims=True)
    d = y - mu
    var = jnp.mean(d * d, axis=-1, keepdims=True)
    out_ref[...] = d * lax.rsqrt(var + 1e-5) * gm_ref[...] + bt_ref[...]


def _mlp(gn2, ee, w1a, w1bc, b1, w2, b2, w3, b3, gm, bt):
    row_spec = lambda w: pl.BlockSpec((BR, w), lambda i: (i, 0))
    full_spec = lambda a: pl.BlockSpec(a.shape, lambda i: (0,) * a.ndim)
    return pl.pallas_call(
        _mlp_block,
        grid=(N_BLOCKS,),
        in_specs=[
            row_spec(NIN),
            row_spec(NEN + NEL),
            full_spec(w1a),
            full_spec(w1bc),
            full_spec(b1),
            full_spec(w2),
            full_spec(b2),
            full_spec(w3),
            full_spec(b3),
            full_spec(gm),
            full_spec(bt),
        ],
        out_specs=row_spec(HID),
        out_shape=jax.ShapeDtypeStruct((R, HID), jnp.float32),
        compiler_params=pltpu.CompilerParams(
            dimension_semantics=("arbitrary",),
        ),
    )(gn2, ee, w1a, w1bc, b1, w2, b2, w3, b3, gm, bt)


NCHUNK = 8
CHW = HID // NCHUNK
NTILES = 16
ROWS_PT = R // NTILES
SB = 1000
NB = ROWS_PT // SB
DPT = DIMSIZE // NTILES


def _sc_body(interim, idx3, zrs, out, idx_all, rows_v, acc, sem_in, sem_sc):
    c = lax.axis_index("c")
    s = lax.axis_index("s")
    pltpu.sync_copy(idx3.at[pl.ds(s * NB, NB)], idx_all)

    for ch in range(NCHUNK):
        f0 = ch * CHW

        @pl.when(c == (ch % 2))
        def _chunk():
            pltpu.sync_copy(zrs, acc.at[pl.ds(s * DPT, DPT)])
            plsc.subcore_barrier()

            def _start(b, buf):
                row0 = (s * NB + b) * SB
                pltpu.async_copy(
                    interim.at[pl.ds(row0, SB), pl.ds(f0, CHW)],
                    rows_v.at[buf], sem_in.at[buf])

            _start(0, 0)

            def _drain_scat(buf):
                for j in range(8):
                    pltpu.make_async_copy(
                        rows_v.at[buf, pl.ds(j * 125, 125)],
                        acc.at[idx_all.at[0, j]],
                        sem_sc.at[buf]).wait()

            def body(b, carry):
                buf = lax.rem(b, 2)

                @pl.when(b >= 1)
                def _dr():
                    _drain_scat(1 - buf)

                @pl.when(b + 1 < NB)
                def _pre():
                    _start(b + 1, 1 - buf)

                pltpu.make_async_copy(
                    interim.at[pl.ds(0, SB), pl.ds(f0, CHW)],
                    rows_v.at[buf], sem_in.at[buf]).wait()
                for j in range(8):
                    pltpu.async_copy(
                        rows_v.at[buf, pl.ds(j * 125, 125)],
                        acc.at[idx_all.at[b, j]],
                        sem_sc.at[buf], add=True)
                return carry

            lax.fori_loop(0, NB, body, 0)
            _drain_scat(lax.rem(NB - 1, 2))
            plsc.subcore_barrier()
            pltpu.sync_copy(
                acc.at[pl.ds(s * DPT, DPT)],
                out.at[0, pl.ds(s * DPT, DPT), pl.ds(f0, CHW)])
            plsc.subcore_barrier()


def _scatter(interim, idx3, zrs):
    mesh = plsc.VectorSubcoreMesh(core_axis_name="c", subcore_axis_name="s")
    return pl.kernel(
        _sc_body,
        mesh=mesh,
        compiler_params=pltpu.CompilerParams(use_tc_tiling_on_sc=False),
        out_type=jax.ShapeDtypeStruct((1, DIMSIZE, HID), jnp.float32),
        scratch_types=[
            pltpu.VMEM((NB, 8, 125), jnp.int32),
            pltpu.VMEM((2, SB, CHW), jnp.float32),
            pltpu.VMEM_SHARED((DIMSIZE, CHW), jnp.float32),
            pltpu.SemaphoreType.DMA((2,)),
            pltpu.SemaphoreType.DMA((2,)),
        ],
    )(interim, idx3, zrs)


def kernel(gatheredNodes, elementNodes, elements, elemConnScatter, maxNelem,
           nodeDim, dimSize, W1, b1, W2, b2, W3, b3, gamma, beta):
    gn2 = gatheredNodes.reshape(R, NIN)
    el_exp = jnp.broadcast_to(elements[:, :, None, :], (1, E, MAXN, NEL))
    ee = jnp.concatenate(
        [elementNodes.reshape(R, NEN), el_exp.reshape(R, NEL)], axis=-1)
    interim = _mlp(gn2, ee, W1[:NIN], W1[NIN:], b1.reshape(1, HID),
                   W2, b2.reshape(1, HID), W3, b3.reshape(1, HID),
                   gamma.reshape(1, HID), beta.reshape(1, HID))
    idx3 = elemConnScatter.reshape(R // SB, 8, 125)
    zrs = jnp.zeros((DPT, CHW), jnp.float32)
    return _scatter(interim, idx3, zrs)


# speedup vs baseline: 4.8606x; 1.3480x over previous
import functools

import jax
import jax.numpy as jnp
from jax import lax
from jax.experimental import pallas as pl
from jax.experimental.pallas import tpu as pltpu
from jax.experimental.pallas import tpu_sc as plsc

NIN = 128
NEN = 16
NEL = 16
HID = 128
E = 100000
MAXN = 4
R = E * MAXN
DIMSIZE = 50000


BR = 4096
N_BLOCKS = -(-R // BR)

_INV_SQRT2 = 0.7071067811865476


def _gelu(x):
    return 0.5 * x * (1.0 + lax.erf(x * _INV_SQRT2))


def _mlp_block(gn_ref, ent_ref, elt_ref, w1a_ref, w1b_ref, w1c_ref, b1_ref,
               w2_ref, b2_ref, w3_ref, b3_ref, gm_ref, bt_ref, out_ref):
    dn = (((0,), (0,)), ((), ()))
    cel = lax.dot_general(elt_ref[...], w1c_ref[...], dn,
                          preferred_element_type=jnp.float32)
    cns = [
        cel + lax.dot_general(ent_ref[n * NEN:(n + 1) * NEN, :], w1b_ref[...],
                              dn, preferred_element_type=jnp.float32)
        for n in range(MAXN)
    ]
    mix = jnp.stack(cns, axis=1).reshape(BR, HID)
    x = (jnp.dot(gn_ref[...], w1a_ref[...], preferred_element_type=jnp.float32)
         + mix + b1_ref[...])
    h = _gelu(x)
    h = _gelu(jnp.dot(h, w2_ref[...], preferred_element_type=jnp.float32)
              + b2_ref[...])
    y = jnp.dot(h, w3_ref[...], preferred_element_type=jnp.float32) + b3_ref[...]
    mu = jnp.mean(y, axis=-1, keepdims=True)
    d = y - mu
    var = jnp.mean(d * d, axis=-1, keepdims=True)
    out_ref[...] = d * lax.rsqrt(var + 1e-5) * gm_ref[...] + bt_ref[...]


def _mlp(gn2, ent, elt, w1a, w1b, w1c, b1, w2, b2, w3, b3, gm, bt):
    BE = BR // MAXN
    row_spec = lambda w: pl.BlockSpec((BR, w), lambda i: (i, 0))
    col_spec = lambda h: pl.BlockSpec((h, BE), lambda i: (0, i))
    full_spec = lambda a: pl.BlockSpec(a.shape, lambda i: (0,) * a.ndim)
    return pl.pallas_call(
        _mlp_block,
        grid=(N_BLOCKS,),
        in_specs=[
            row_spec(NIN),
            col_spec(MAXN * NEN),
            col_spec(NEL),
            full_spec(w1a),
            full_spec(w1b),
            full_spec(w1c),
            full_spec(b1),
            full_spec(w2),
            full_spec(b2),
            full_spec(w3),
            full_spec(b3),
            full_spec(gm),
            full_spec(bt),
        ],
        out_specs=row_spec(HID),
        out_shape=jax.ShapeDtypeStruct((R, HID), jnp.float32),
        compiler_params=pltpu.CompilerParams(
            dimension_semantics=("arbitrary",),
        ),
    )(gn2, ent, elt, w1a, w1b, w1c, b1, w2, b2, w3, b3, gm, bt)


NCHUNK = 8
CHW = HID // NCHUNK
NTILES = 16
ROWS_PT = R // NTILES
SB = 1000
NB = ROWS_PT // SB
DPT = DIMSIZE // NTILES


def _sc_body(interim, idx3, zrs, out, idx_all, rows_v, acc, sem_in, sem_sc):
    c = lax.axis_index("c")
    s = lax.axis_index("s")
    pltpu.sync_copy(idx3.at[pl.ds(s * NB, NB)], idx_all)

    for ch in range(NCHUNK):
        f0 = ch * CHW

        @pl.when(c == (ch % 2))
        def _chunk():
            pltpu.sync_copy(zrs, acc.at[pl.ds(s * DPT, DPT)])
            plsc.subcore_barrier()

            def _start(b, buf):
                row0 = (s * NB + b) * SB
                pltpu.async_copy(
                    interim.at[pl.ds(row0, SB), pl.ds(f0, CHW)],
                    rows_v.at[buf], sem_in.at[buf])

            _start(0, 0)

            def _drain_scat(buf):
                for j in range(8):
                    pltpu.make_async_copy(
                        rows_v.at[buf, pl.ds(j * 125, 125)],
                        acc.at[idx_all.at[0, j]],
                        sem_sc.at[buf]).wait()

            def body(b, carry):
                buf = lax.rem(b, 2)

                @pl.when(b >= 1)
                def _dr():
                    _drain_scat(1 - buf)

                @pl.when(b + 1 < NB)
                def _pre():
                    _start(b + 1, 1 - buf)

                pltpu.make_async_copy(
                    interim.at[pl.ds(0, SB), pl.ds(f0, CHW)],
                    rows_v.at[buf], sem_in.at[buf]).wait()
                for j in range(8):
                    pltpu.async_copy(
                        rows_v.at[buf, pl.ds(j * 125, 125)],
                        acc.at[idx_all.at[b, j]],
                        sem_sc.at[buf], add=True)
                return carry

            lax.fori_loop(0, NB, body, 0)
            _drain_scat(lax.rem(NB - 1, 2))
            plsc.subcore_barrier()
            pltpu.sync_copy(
                acc.at[pl.ds(s * DPT, DPT)],
                out.at[0, pl.ds(s * DPT, DPT), pl.ds(f0, CHW)])
            plsc.subcore_barrier()


def _scatter(interim, idx3, zrs):
    mesh = plsc.VectorSubcoreMesh(core_axis_name="c", subcore_axis_name="s")
    return pl.kernel(
        _sc_body,
        mesh=mesh,
        compiler_params=pltpu.CompilerParams(use_tc_tiling_on_sc=False),
        out_type=jax.ShapeDtypeStruct((1, DIMSIZE, HID), jnp.float32),
        scratch_types=[
            pltpu.VMEM((NB, 8, 125), jnp.int32),
            pltpu.VMEM((2, SB, CHW), jnp.float32),
            pltpu.VMEM_SHARED((DIMSIZE, CHW), jnp.float32),
            pltpu.SemaphoreType.DMA((2,)),
            pltpu.SemaphoreType.DMA((2,)),
        ],
    )(interim, idx3, zrs)


def kernel(gatheredNodes, elementNodes, elements, elemConnScatter, maxNelem,
           nodeDim, dimSize, W1, b1, W2, b2, W3, b3, gamma, beta):
    gn2 = gatheredNodes.reshape(R, NIN)
    ent = jnp.transpose(elementNodes, (0, 2, 3, 1)).reshape(MAXN * NEN, E)
    elt = jnp.transpose(elements, (0, 2, 1)).reshape(NEL, E)
    interim = _mlp(gn2, ent, elt, W1[:NIN], W1[NIN:NIN + NEN],
                   W1[NIN + NEN:], b1.reshape(1, HID),
                   W2, b2.reshape(1, HID), W3, b3.reshape(1, HID),
                   gamma.reshape(1, HID), beta.reshape(1, HID))
    idx3 = elemConnScatter.reshape(R // SB, 8, 125)
    zrs = jnp.zeros((DPT, CHW), jnp.float32)
    return _scatter(interim, idx3, zrs)


# speedup vs baseline: 4.8707x; 1.0021x over previous
import jax
import jax.numpy as jnp
from jax import lax
from jax.experimental import pallas as pl
from jax.experimental.pallas import tpu as pltpu
from jax.experimental.pallas import tpu_sc as plsc

NIN = 128
NEN = 16
NEL = 16
HID = 128
E = 100000
MAXN = 4
R = E * MAXN
DIMSIZE = 50000

BR = 4096
BE = BR // MAXN
R_PAD = 409600
SPLITS = ((0, 48, 48 * BR), (48, 50, 52 * BR))

_INV_SQRT2 = 0.7071067811865476


def _gelu(x):
    return 0.5 * x * (1.0 + lax.erf(x * _INV_SQRT2))


def _mlp_block(gn_ref, ent_ref, elt_ref, w1a_ref, w1b_ref, w1c_ref, b1_ref,
               w2_ref, b2_ref, w3_ref, b3_ref, gm_ref, bt_ref, out_ref):
    dn = (((0,), (0,)), ((), ()))
    cel = lax.dot_general(elt_ref[...], w1c_ref[...], dn,
                          preferred_element_type=jnp.float32)
    cns = [
        cel + lax.dot_general(ent_ref[n * NEN:(n + 1) * NEN, :], w1b_ref[...],
                              dn, preferred_element_type=jnp.float32)
        for n in range(MAXN)
    ]
    mix = jnp.stack(cns, axis=1).reshape(BR, HID)
    x = (jnp.dot(gn_ref[...], w1a_ref[...], preferred_element_type=jnp.float32)
         + mix + b1_ref[...])
    h = _gelu(x)
    h = _gelu(jnp.dot(h, w2_ref[...], preferred_element_type=jnp.float32)
              + b2_ref[...])
    y = jnp.dot(h, w3_ref[...], preferred_element_type=jnp.float32) + b3_ref[...]
    mu = jnp.mean(y, axis=-1, keepdims=True)
    d = y - mu
    var = jnp.mean(d * d, axis=-1, keepdims=True)
    out_ref[...] = d * lax.rsqrt(var + 1e-5) * gm_ref[...] + bt_ref[...]


def _mlp(gn2, ent, elt, w1a, w1b, w1c, b1, w2, b2, w3, b3, gm, bt,
         b0, nblk, out_rows):
    row_spec = lambda w: pl.BlockSpec((BR, w), lambda i: (i + b0, 0))
    out_spec = pl.BlockSpec((BR, HID), lambda i: (i, 0))
    col_spec = lambda h: pl.BlockSpec((h, BE), lambda i: (0, i + b0))
    full_spec = lambda a: pl.BlockSpec(a.shape, lambda i: (0,) * a.ndim)
    return pl.pallas_call(
        _mlp_block,
        grid=(nblk,),
        in_specs=[
            row_spec(NIN),
            col_spec(MAXN * NEN),
            col_spec(NEL),
            full_spec(w1a),
            full_spec(w1b),
            full_spec(w1c),
            full_spec(b1),
            full_spec(w2),
            full_spec(b2),
            full_spec(w3),
            full_spec(b3),
            full_spec(gm),
            full_spec(bt),
        ],
        out_specs=out_spec,
        out_shape=jax.ShapeDtypeStruct((out_rows, HID), jnp.float32),
        compiler_params=pltpu.CompilerParams(
            dimension_semantics=("arbitrary",),
        ),
    )(gn2, ent, elt, w1a, w1b, w1c, b1, w2, b2, w3, b3, gm, bt)


NCHUNK = 8
CHW = HID // NCHUNK
NTILES = 16
SB = 1024
DPT = DIMSIZE // NTILES
ACCD = DIMSIZE + 8


def _make_sc_body(offb, nb, first):
    def _sc_body(interim, idx3, init, out, idx_all, rows_v, acc,
                 sem_in, sem_sc):
        c = lax.axis_index("c")
        s = lax.axis_index("s")
        pltpu.sync_copy(idx3.at[pl.ds(offb + s * nb, nb)], idx_all)

        for ch in range(NCHUNK):
            f0 = ch * CHW

            @pl.when(c == (ch % 2))
            def _chunk():
                if first:
                    pltpu.sync_copy(init, acc.at[pl.ds(s * DPT, DPT)])
                else:
                    pltpu.sync_copy(
                        init.at[0, pl.ds(s * DPT, DPT), pl.ds(f0, CHW)],
                        acc.at[pl.ds(s * DPT, DPT)])
                plsc.subcore_barrier()

                def _start(b, buf):
                    row0 = (s * nb + b) * SB
                    pltpu.async_copy(
                        interim.at[pl.ds(row0, SB), pl.ds(f0, CHW)],
                        rows_v.at[buf], sem_in.at[buf])

                _start(0, 0)

                def _drain_scat(buf):
                    for j in range(8):
                        pltpu.make_async_copy(
                            rows_v.at[buf, pl.ds(j * 128, 128)],
                            acc.at[idx_all.at[0, j]],
                            sem_sc.at[buf]).wait()

                def body(b, carry):
                    buf = lax.rem(b, 2)

                    @pl.when(b >= 1)
                    def _dr():
                        _drain_scat(1 - buf)

                    @pl.when(b + 1 < nb)
                    def _pre():
                        _start(b + 1, 1 - buf)

                    pltpu.make_async_copy(
                        interim.at[pl.ds(0, SB), pl.ds(f0, CHW)],
                        rows_v.at[buf], sem_in.at[buf]).wait()
                    for j in range(8):
                        pltpu.async_copy(
                            rows_v.at[buf, pl.ds(j * 128, 128)],
                            acc.at[idx_all.at[b, j]],
                            sem_sc.at[buf], add=True)
                    return carry

                lax.fori_loop(0, nb, body, 0)
                _drain_scat(lax.rem(nb - 1, 2))
                plsc.subcore_barrier()
                pltpu.sync_copy(
                    acc.at[pl.ds(s * DPT, DPT)],
                    out.at[0, pl.ds(s * DPT, DPT), pl.ds(f0, CHW)])
                plsc.subcore_barrier()

    return _sc_body


def _scatter(interim, idx3, init, offb, nb, first):
    mesh = plsc.VectorSubcoreMesh(core_axis_name="c", subcore_axis_name="s")
    return pl.kernel(
        _make_sc_body(offb, nb, first),
        mesh=mesh,
        compiler_params=pltpu.CompilerParams(use_tc_tiling_on_sc=False),
        out_type=jax.ShapeDtypeStruct((1, DIMSIZE, HID), jnp.float32),
        scratch_types=[
            pltpu.VMEM((nb, 8, 128), jnp.int32),
            pltpu.VMEM((2, SB, CHW), jnp.float32),
            pltpu.VMEM_SHARED((ACCD, CHW), jnp.float32),
            pltpu.SemaphoreType.DMA((2,)),
            pltpu.SemaphoreType.DMA((2,)),
        ],
    )(interim, idx3, init)


def kernel(gatheredNodes, elementNodes, elements, elemConnScatter, maxNelem,
           nodeDim, dimSize, W1, b1, W2, b2, W3, b3, gamma, beta):
    gn2 = gatheredNodes.reshape(R, NIN)
    ent = jnp.transpose(elementNodes, (0, 2, 3, 1)).reshape(MAXN * NEN, E)
    elt = jnp.transpose(elements, (0, 2, 1)).reshape(NEL, E)
    w = (W1[:NIN], W1[NIN:NIN + NEN], W1[NIN + NEN:], b1.reshape(1, HID),
         W2, b2.reshape(1, HID), W3, b3.reshape(1, HID),
         gamma.reshape(1, HID), beta.reshape(1, HID))
    pad_idx = DIMSIZE + (jnp.arange(R_PAD - R, dtype=jnp.int32) % 8)
    idx3 = jnp.concatenate([elemConnScatter, pad_idx]).reshape(
        R_PAD // SB, 8, 128)
    zrs = jnp.zeros((DPT, CHW), jnp.float32)

    (b0a, nba, rowsa), (b0b, nbb, rowsb) = SPLITS
    interim_a = _mlp(gn2, ent, elt, *w, b0=b0a, nblk=nba, out_rows=rowsa)
    interim_b = _mlp(gn2, ent, elt, *w, b0=b0b, nblk=nbb, out_rows=rowsb)
    nb_a = rowsa // (NTILES * SB)
    nb_b = rowsb // (NTILES * SB)
    part = _scatter(interim_a, idx3, zrs, offb=0, nb=nb_a, first=True)
    return _scatter(interim_b, idx3, part, offb=rowsa // SB, nb=nb_b,
                    first=False)


# speedup vs baseline: 5.0572x; 1.0383x over previous
import jax
import jax.numpy as jnp
from jax import lax
from jax.experimental import pallas as pl
from jax.experimental.pallas import tpu as pltpu
from jax.experimental.pallas import tpu_sc as plsc

NIN = 128
NEN = 16
NEL = 16
HID = 128
E = 100000
MAXN = 4
R = E * MAXN
DIMSIZE = 50000

BR = 4096
BE = BR // MAXN
R_PAD = 409600
SPLITS = ((0, 48, 48 * BR), (48, 50, 52 * BR))

_INV_SQRT2 = 0.7071067811865476


def _gelu(x):
    return 0.5 * x * (1.0 + lax.erf(x * _INV_SQRT2))


def _mlp_block(gn_ref, ent_ref, elt_ref, w1a_ref, wen_ref, b1_ref,
               w2_ref, b2_ref, w3_ref, b3_ref, gm_ref, bt_ref, out_ref):
    dn = (((0,), (0,)), ((), ()))
    a = jnp.concatenate([ent_ref[...], elt_ref[...]], axis=0)
    z = lax.dot_general(a, wen_ref[...], dn,
                        preferred_element_type=jnp.float32)
    cns = [z[:, n * HID:(n + 1) * HID] for n in range(MAXN)]
    mix = jnp.stack(cns, axis=1).reshape(BR, HID)
    x = (jnp.dot(gn_ref[...], w1a_ref[...], preferred_element_type=jnp.float32)
         + mix + b1_ref[...])
    h = _gelu(x)
    h = _gelu(jnp.dot(h, w2_ref[...], preferred_element_type=jnp.float32)
              + b2_ref[...])
    y = jnp.dot(h, w3_ref[...], preferred_element_type=jnp.float32) + b3_ref[...]
    mu = jnp.mean(y, axis=-1, keepdims=True)
    d = y - mu
    var = jnp.mean(d * d, axis=-1, keepdims=True)
    out_ref[...] = d * lax.rsqrt(var + 1e-5) * gm_ref[...] + bt_ref[...]


def _mlp(gn2, ent, elt, w1a, wen, b1, w2, b2, w3, b3, gm, bt,
         b0, nblk, out_rows):
    row_spec = lambda w: pl.BlockSpec((BR, w), lambda i: (i + b0, 0))
    out_spec = pl.BlockSpec((BR, HID), lambda i: (i, 0))
    col_spec = lambda h: pl.BlockSpec((h, BE), lambda i: (0, i + b0))
    full_spec = lambda a: pl.BlockSpec(a.shape, lambda i: (0,) * a.ndim)
    return pl.pallas_call(
        _mlp_block,
        grid=(nblk,),
        in_specs=[
            row_spec(NIN),
            col_spec(MAXN * NEN),
            col_spec(NEL),
            full_spec(w1a),
            full_spec(wen),
            full_spec(b1),
            full_spec(w2),
            full_spec(b2),
            full_spec(w3),
            full_spec(b3),
            full_spec(gm),
            full_spec(bt),
        ],
        out_specs=out_spec,
        out_shape=jax.ShapeDtypeStruct((out_rows, HID), jnp.float32),
        compiler_params=pltpu.CompilerParams(
            dimension_semantics=("arbitrary",),
        ),
    )(gn2, ent, elt, w1a, wen, b1, w2, b2, w3, b3, gm, bt)


NCHUNK = 8
CHW = HID // NCHUNK
NTILES = 16
SB = 1024
DPT = DIMSIZE // NTILES
ACCD = DIMSIZE + 8


def _make_sc_body(offb, nb, first):
    def _sc_body(interim, idx3, init, out, idx_all, rows_v, acc,
                 sem_in, sem_sc):
        c = lax.axis_index("c")
        s = lax.axis_index("s")
        pltpu.sync_copy(idx3.at[pl.ds(offb + s * nb, nb)], idx_all)

        for ch in range(NCHUNK):
            f0 = ch * CHW

            @pl.when(c == (ch % 2))
            def _chunk():
                if first:
                    pltpu.sync_copy(init, acc.at[pl.ds(s * DPT, DPT)])
                else:
                    pltpu.sync_copy(
                        init.at[0, pl.ds(s * DPT, DPT), pl.ds(f0, CHW)],
                        acc.at[pl.ds(s * DPT, DPT)])
                plsc.subcore_barrier()

                def _start(b, buf):
                    row0 = (s * nb + b) * SB
                    pltpu.async_copy(
                        interim.at[pl.ds(row0, SB), pl.ds(f0, CHW)],
                        rows_v.at[buf], sem_in.at[buf])

                _start(0, 0)

                def _drain_scat(buf):
                    for j in range(8):
                        pltpu.make_async_copy(
                            rows_v.at[buf, pl.ds(j * 128, 128)],
                            acc.at[idx_all.at[0, j]],
                            sem_sc.at[buf]).wait()

                def body(b, carry):
                    buf = lax.rem(b, 2)

                    @pl.when(b >= 1)
                    def _dr():
                        _drain_scat(1 - buf)

                    @pl.when(b + 1 < nb)
                    def _pre():
                        _start(b + 1, 1 - buf)

                    pltpu.make_async_copy(
                        interim.at[pl.ds(0, SB), pl.ds(f0, CHW)],
                        rows_v.at[buf], sem_in.at[buf]).wait()
                    for j in range(8):
                        pltpu.async_copy(
                            rows_v.at[buf, pl.ds(j * 128, 128)],
                            acc.at[idx_all.at[b, j]],
                            sem_sc.at[buf], add=True)
                    return carry

                lax.fori_loop(0, nb, body, 0)
                _drain_scat(lax.rem(nb - 1, 2))
                plsc.subcore_barrier()
                pltpu.sync_copy(
                    acc.at[pl.ds(s * DPT, DPT)],
                    out.at[0, pl.ds(s * DPT, DPT), pl.ds(f0, CHW)])
                plsc.subcore_barrier()

    return _sc_body


def _scatter(interim, idx3, init, offb, nb, first):
    mesh = plsc.VectorSubcoreMesh(core_axis_name="c", subcore_axis_name="s")
    return pl.kernel(
        _make_sc_body(offb, nb, first),
        mesh=mesh,
        compiler_params=pltpu.CompilerParams(use_tc_tiling_on_sc=False),
        out_type=jax.ShapeDtypeStruct((1, DIMSIZE, HID), jnp.float32),
        scratch_types=[
            pltpu.VMEM((nb, 8, 128), jnp.int32),
            pltpu.VMEM((2, SB, CHW), jnp.float32),
            pltpu.VMEM_SHARED((ACCD, CHW), jnp.float32),
            pltpu.SemaphoreType.DMA((2,)),
            pltpu.SemaphoreType.DMA((2,)),
        ],
    )(interim, idx3, init)


def kernel(gatheredNodes, elementNodes, elements, elemConnScatter, maxNelem,
           nodeDim, dimSize, W1, b1, W2, b2, W3, b3, gamma, beta):
    gn2 = gatheredNodes.reshape(R, NIN)
    ent = jnp.transpose(elementNodes, (0, 2, 3, 1)).reshape(MAXN * NEN, E)
    elt = jnp.transpose(elements, (0, 2, 1)).reshape(NEL, E)
    wen = jnp.concatenate(
        [jnp.kron(jnp.eye(MAXN, dtype=jnp.float32), W1[NIN:NIN + NEN]),
         jnp.tile(W1[NIN + NEN:], (1, MAXN))], axis=0)
    w = (W1[:NIN], wen, b1.reshape(1, HID),
         W2, b2.reshape(1, HID), W3, b3.reshape(1, HID),
         gamma.reshape(1, HID), beta.reshape(1, HID))
    pad_idx = DIMSIZE + (jnp.arange(R_PAD - R, dtype=jnp.int32) % 8)
    idx3 = jnp.concatenate([elemConnScatter, pad_idx]).reshape(
        R_PAD // SB, 8, 128)
    zrs = jnp.zeros((DPT, CHW), jnp.float32)

    (b0a, nba, rowsa), (b0b, nbb, rowsb) = SPLITS
    interim_a = _mlp(gn2, ent, elt, *w, b0=b0a, nblk=nba, out_rows=rowsa)
    interim_b = _mlp(gn2, ent, elt, *w, b0=b0b, nblk=nbb, out_rows=rowsb)
    nb_a = rowsa // (NTILES * SB)
    nb_b = rowsb // (NTILES * SB)
    part = _scatter(interim_a, idx3, zrs, offb=0, nb=nb_a, first=True)
    return _scatter(interim_b, idx3, part, offb=rowsa // SB, nb=nb_b,
                    first=False)
